# baseline (speedup 1.0000x reference)
import jax
import jax.numpy as jnp
from jax import lax
from jax.experimental import pallas as pl
from jax.experimental.pallas import tpu as pltpu
from jax.experimental.pallas import tpu_sc as plsc

N = 10000
E = 320000
D = 128
NC = 2
NS = 16
NW = NC * NS
NP = 10240
EPT = E // NW
C = 80
NCHUNK = EPT // C
G = 5
K = NCHUNK // G
RPT = NP // NS


def _make_sc_agg(with_counts: bool):
    out_type = [jax.ShapeDtypeStruct((NC, NP, D), jnp.float32)]
    scratch = [
        pltpu.VMEM_SHARED((NP, D), jnp.float32),
        pltpu.VMEM((K, C), jnp.int32),
        pltpu.VMEM((K, C), jnp.int32),
        pltpu.VMEM((K, C), jnp.int32),
        pltpu.VMEM((K, C), jnp.int32),
        pltpu.VMEM((C, D), jnp.float32),
        pltpu.VMEM((C, D), jnp.float32),
        pltpu.SemaphoreType.DMA,
        pltpu.SemaphoreType.DMA,
        pltpu.SemaphoreType.DMA,
        pltpu.SemaphoreType.DMA,
        pltpu.SemaphoreType.DMA,
        pltpu.SemaphoreType.DMA,
    ]
    if with_counts:
        out_type.append(jax.ShapeDtypeStruct((NW * NP,), jnp.float32))
        scratch.append(pltpu.VMEM((NP,), jnp.float32))

    mesh = plsc.VectorSubcoreMesh(core_axis_name="c", subcore_axis_name="s")

    def body(src_hbm, dst_hbm, x_hbm, agg_out, *rest):
        if with_counts:
            (cnt_out, shared_agg, src0, src1, dst0, dst1, rows_a, rows_b,
             sem_ga, sem_gb, sem_sa, sem_sb, sem_i, sem_z, cnt_v) = rest
        else:
            (shared_agg, src0, src1, dst0, dst1, rows_a, rows_b,
             sem_ga, sem_gb, sem_sa, sem_sb, sem_i, sem_z) = rest
        srcbuf = [src0, src1]
        dstbuf = [dst0, dst1]
        cid = lax.axis_index("c")
        sid = lax.axis_index("s")
        wid = sid * NC + cid

        g0s = pltpu.make_async_copy(src_hbm.at[wid, 0], src0, sem_i)
        g0d = pltpu.make_async_copy(dst_hbm.at[wid, 0], dst0, sem_i)
        g0s.start()
        g0d.start()

        z16 = jnp.zeros((16,), jnp.float32)

        def zrow(r, carry):
            for k in range(D // 16):
                rows_a[r, pl.ds(k * 16, 16)] = z16
                rows_b[r, pl.ds(k * 16, 16)] = z16
            return carry
        lax.fori_loop(0, C, zrow, 0)

        base_r = sid * RPT
        zcps = []
        for j in range(RPT // C):
            zsrc = rows_a if j % 2 == 0 else rows_b
            cp = pltpu.make_async_copy(
                zsrc, shared_agg.at[pl.ds(base_r + j * C, C)], sem_z)
            cp.start()
            zcps.append(cp)

        if with_counts:
            def zcnt(i, carry):
                cnt_v[pl.ds(i * 16, 16)] = z16
                return carry
            lax.fori_loop(0, NP // 16, zcnt, 0)

        for cp in zcps:
            cp.wait()
        g0s.wait()
        g0d.wait()
        pltpu.async_copy(x_hbm.at[src0.at[0]], rows_a, sem_ga)
        plsc.subcore_barrier()

        ones16 = jnp.ones((16,), jnp.float32)

        def counts(dst_i, j):
            if with_counts:
                for k in range(C // 16):
                    idx16 = dst_i[j, pl.ds(k * 16, 16)]
                    plsc.addupdate_scatter(cnt_v, [idx16], ones16)

        def step(src_c, dst_c, j, rows, semg, rows_o, semg_o, sems, sems_o,
                 pre_idx, prev_j):
            pltpu.make_async_copy(x_hbm.at[src_c.at[j]], rows, semg).wait()
            counts(dst_c, j)
            if prev_j is not None:
                pltpu.make_async_copy(
                    rows_o, shared_agg.at[dst_c.at[prev_j]], sems_o).wait()
            if pre_idx is not None:
                pltpu.async_copy(x_hbm.at[pre_idx], rows_o, semg_o)
            pltpu.async_copy(rows, shared_agg.at[dst_c.at[j]], sems, add=True)

        for g in range(G):
            src_c, dst_c = srcbuf[g % 2], dstbuf[g % 2]
            src_n, dst_n = srcbuf[(g + 1) % 2], dstbuf[(g + 1) % 2]
            if g % 2 == 0:
                r0, sg0, ss0 = rows_a, sem_ga, sem_sa
                r1, sg1, ss1 = rows_b, sem_gb, sem_sb
            else:
                r0, sg0, ss0 = rows_b, sem_gb, sem_sb
                r1, sg1, ss1 = rows_a, sem_ga, sem_sa
            if g > 0:
                pltpu.make_async_copy(
                    r1, shared_agg.at[dst_n.at[K - 1]], ss1).wait()
            if g + 1 < G:
                gns = pltpu.make_async_copy(src_hbm.at[wid, g + 1], src_n,
                                            sem_i)
                gnd = pltpu.make_async_copy(dst_hbm.at[wid, g + 1], dst_n,
                                            sem_i)
                gns.start()
                gnd.start()

            step(src_c, dst_c, 0, r0, sg0, r1, sg1, ss0, ss1,
                 src_c.at[1], None)
            step(src_c, dst_c, 1, r1, sg1, r0, sg0, ss1, ss0,
                 src_c.at[2], 0)

            def pair(p, carry, src_c=src_c, dst_c=dst_c, r0=r0, sg0=sg0,
                     ss0=ss0, r1=r1, sg1=sg1, ss1=ss1):
                j0 = 2 * p
                step(src_c, dst_c, j0, r0, sg0, r1, sg1, ss0, ss1,
                     src_c.at[j0 + 1], j0 - 1)
                step(src_c, dst_c, j0 + 1, r1, sg1, r0, sg0, ss1, ss0,
                     src_c.at[j0 + 2], j0)
                return carry
            lax.fori_loop(1, K // 2, pair, 0)
            if g + 1 < G:
                gns.wait()
                gnd.wait()
                step(src_c, dst_c, K - 1, r0, sg0, r1, sg1, ss0, ss1,
                     src_n.at[0], K - 2)
            else:
                step(src_c, dst_c, K - 1, r0, sg0, r1, sg1, ss0, ss1,
                     None, K - 2)
                pltpu.make_async_copy(
                    r0, shared_agg.at[dst_c.at[K - 1]], ss0).wait()

        plsc.subcore_barrier()
        pltpu.sync_copy(shared_agg.at[pl.ds(base_r, RPT)],
                        agg_out.at[cid, pl.ds(base_r, RPT)])
        if with_counts:
            pltpu.sync_copy(cnt_v, cnt_out.at[pl.ds(wid * NP, NP)])

    return pl.kernel(
        body, out_type=out_type, mesh=mesh, scratch_types=scratch,
        compiler_params=pltpu.CompilerParams(needs_layout_passes=False))


_sc_agg_counts = _make_sc_agg(True)
_sc_agg = _make_sc_agg(False)


BN = 400


def _dotT(a, w):
    return lax.dot_general(a, w, (((1,), (1,)), ((), ())),
                           preferred_element_type=jnp.float32)


def _dense_xr_body(x_ref, wr_ref, b_ref, o_ref):
    o_ref[...] = _dotT(x_ref[...], wr_ref[...]) + b_ref[...]


_dense_xr = pl.pallas_call(
    _dense_xr_body,
    grid=(N // BN,),
    in_specs=[
        pl.BlockSpec((BN, D), lambda i: (i, 0)),
        pl.BlockSpec((D, D), lambda i: (0, 0)),
        pl.BlockSpec((1, D), lambda i: (0, 0)),
    ],
    out_specs=pl.BlockSpec((BN, D), lambda i: (i, 0)),
    out_shape=jax.ShapeDtypeStruct((N, D), jnp.float32),
)


def _dense0_body(agg_ref, cnt_ref, xr_ref, wl_ref, o_ref):
    agg = agg_ref[0] + agg_ref[1]
    cnt = jnp.sum(cnt_ref[...], axis=1)
    mean = agg / jnp.maximum(cnt, 1.0)[:, None]
    o_ref[...] = jnp.maximum(_dotT(mean, wl_ref[...]) + xr_ref[...], 0.0)


_dense0 = pl.pallas_call(
    _dense0_body,
    grid=(N // BN,),
    in_specs=[
        pl.BlockSpec((NC, BN, D), lambda i: (0, i, 0)),
        pl.BlockSpec((BN, NW), lambda i: (i, 0)),
        pl.BlockSpec((BN, D), lambda i: (i, 0)),
        pl.BlockSpec((D, D), lambda i: (0, 0)),
    ],
    out_specs=pl.BlockSpec((BN, D), lambda i: (i, 0)),
    out_shape=jax.ShapeDtypeStruct((N, D), jnp.float32),
)


def _dense1_body(agg_ref, cnt_ref, xr_ref, wl_ref,
                 wp1_ref, bp1_ref, wp2_ref, bp2_ref, o_ref):
    agg = agg_ref[0] + agg_ref[1]
    cnt = jnp.sum(cnt_ref[...], axis=1)
    mean = agg / jnp.maximum(cnt, 1.0)[:, None]
    h1 = jnp.maximum(_dotT(mean, wl_ref[...]) + xr_ref[...], 0.0)
    p = _dotT(h1, wp1_ref[...]) + bp1_ref[...]
    o_ref[...] = _dotT(p, wp2_ref[...]) + bp2_ref[...]


_dense1 = pl.pallas_call(
    _dense1_body,
    grid=(N // BN,),
    in_specs=[
        pl.BlockSpec((NC, BN, D), lambda i: (0, i, 0)),
        pl.BlockSpec((BN, NW), lambda i: (i, 0)),
        pl.BlockSpec((BN, D), lambda i: (i, 0)),
        pl.BlockSpec((D, D), lambda i: (0, 0)),
        pl.BlockSpec((D, D), lambda i: (0, 0)),
        pl.BlockSpec((1, D), lambda i: (0, 0)),
        pl.BlockSpec((D, D), lambda i: (0, 0)),
        pl.BlockSpec((1, D), lambda i: (0, 0)),
    ],
    out_specs=pl.BlockSpec((BN, D), lambda i: (i, 0)),
    out_shape=jax.ShapeDtypeStruct((N, D), jnp.float32),
)


def kernel(x, edge_index, Wl0, bl0, Wr0, Wl1, bl1, Wr1, Wp1, bp1, Wp2, bp2):
    src = edge_index[0].reshape(NW, G, K, C)
    dst = edge_index[1].reshape(NW, G, K, C)
    xr0 = _dense_xr(x, Wr0, bl0.reshape(1, D))
    agg0, cnt_flat = _sc_agg_counts(src, dst, x)
    cnt_t = cnt_flat.reshape(NW, NP).T
    h0 = _dense0(agg0, cnt_t, xr0, Wl0)
    xr1 = _dense_xr(h0, Wr1, bl1.reshape(1, D))
    (agg1,) = _sc_agg(src, dst, h0)
    return _dense1(agg1, cnt_t, xr1, Wl1,
                   Wp1, bp1.reshape(1, D), Wp2, bp2.reshape(1, D))


# speedup vs baseline: 1.0308x; 1.0308x over previous
import jax
import jax.numpy as jnp
from jax import lax
from jax.experimental import pallas as pl
from jax.experimental.pallas import tpu as pltpu
from jax.experimental.pallas import tpu_sc as plsc

N = 10000
E = 320000
D = 128
NC = 2
NS = 16
NW = NC * NS
NP = 10240
EPT = E // NW
C = 80
NCHUNK = EPT // C
G = 5
K = NCHUNK // G
RPT = NP // NS


def _make_sc_agg(with_counts: bool):
    out_type = [jax.ShapeDtypeStruct((NC, NP, D), jnp.float32)]
    scratch = [
        pltpu.VMEM_SHARED((NP, D), jnp.float32),
        pltpu.VMEM((K, C), jnp.int32),
        pltpu.VMEM((K, C), jnp.int32),
        pltpu.VMEM((K, C), jnp.int32),
        pltpu.VMEM((K, C), jnp.int32),
        pltpu.VMEM((C, D), jnp.float32),
        pltpu.VMEM((C, D), jnp.float32),
        pltpu.SemaphoreType.DMA,
        pltpu.SemaphoreType.DMA,
        pltpu.SemaphoreType.DMA,
        pltpu.SemaphoreType.DMA,
        pltpu.SemaphoreType.DMA,
        pltpu.SemaphoreType.DMA,
    ]
    if with_counts:
        out_type.append(jax.ShapeDtypeStruct((NW * NP,), jnp.float32))
        scratch.append(pltpu.VMEM((NP,), jnp.float32))

    mesh = plsc.VectorSubcoreMesh(core_axis_name="c", subcore_axis_name="s")

    def body(e_hbm, x_hbm, agg_out, *rest):
        if with_counts:
            (cnt_out, shared_agg, src0, src1, dst0, dst1, rows_a, rows_b,
             sem_ga, sem_gb, sem_sa, sem_sb, sem_i, sem_z, cnt_v) = rest
        else:
            (shared_agg, src0, src1, dst0, dst1, rows_a, rows_b,
             sem_ga, sem_gb, sem_sa, sem_sb, sem_i, sem_z) = rest
        srcbuf = [src0, src1]
        dstbuf = [dst0, dst1]
        cid = lax.axis_index("c")
        sid = lax.axis_index("s")
        wid = sid * NC + cid

        g0s = pltpu.make_async_copy(e_hbm.at[0, wid, 0], src0, sem_i)
        g0d = pltpu.make_async_copy(e_hbm.at[1, wid, 0], dst0, sem_i)
        g0s.start()
        g0d.start()

        z16 = jnp.zeros((16,), jnp.float32)

        def zrow(r, carry):
            for k in range(D // 16):
                rows_a[r, pl.ds(k * 16, 16)] = z16
                rows_b[r, pl.ds(k * 16, 16)] = z16
            return carry
        lax.fori_loop(0, C, zrow, 0)

        base_r = sid * RPT
        zcps = []
        for j in range(RPT // C):
            zsrc = rows_a if j % 2 == 0 else rows_b
            cp = pltpu.make_async_copy(
                zsrc, shared_agg.at[pl.ds(base_r + j * C, C)], sem_z)
            cp.start()
            zcps.append(cp)

        if with_counts:
            def zcnt(i, carry):
                cnt_v[pl.ds(i * 16, 16)] = z16
                return carry
            lax.fori_loop(0, NP // 16, zcnt, 0)

        for cp in zcps:
            cp.wait()
        g0s.wait()
        g0d.wait()
        pltpu.async_copy(x_hbm.at[src0.at[0]], rows_a, sem_ga)
        plsc.subcore_barrier()

        ones16 = jnp.ones((16,), jnp.float32)

        def counts(dst_i, j):
            if with_counts:
                for k in range(C // 16):
                    idx16 = dst_i[j, pl.ds(k * 16, 16)]
                    plsc.addupdate_scatter(cnt_v, [idx16], ones16)

        def step(src_c, dst_c, j, rows, semg, rows_o, semg_o, sems, sems_o,
                 pre_idx, prev_j):
            pltpu.make_async_copy(x_hbm.at[src_c.at[j]], rows, semg).wait()
            counts(dst_c, j)
            if prev_j is not None:
                pltpu.make_async_copy(
                    rows_o, shared_agg.at[dst_c.at[prev_j]], sems_o).wait()
            if pre_idx is not None:
                pltpu.async_copy(x_hbm.at[pre_idx], rows_o, semg_o)
            pltpu.async_copy(rows, shared_agg.at[dst_c.at[j]], sems, add=True)

        for g in range(G):
            src_c, dst_c = srcbuf[g % 2], dstbuf[g % 2]
            src_n, dst_n = srcbuf[(g + 1) % 2], dstbuf[(g + 1) % 2]
            if g % 2 == 0:
                r0, sg0, ss0 = rows_a, sem_ga, sem_sa
                r1, sg1, ss1 = rows_b, sem_gb, sem_sb
            else:
                r0, sg0, ss0 = rows_b, sem_gb, sem_sb
                r1, sg1, ss1 = rows_a, sem_ga, sem_sa
            if g > 0:
                pltpu.make_async_copy(
                    r1, shared_agg.at[dst_n.at[K - 1]], ss1).wait()
            if g + 1 < G:
                gns = pltpu.make_async_copy(e_hbm.at[0, wid, g + 1], src_n,
                                            sem_i)
                gnd = pltpu.make_async_copy(e_hbm.at[1, wid, g + 1], dst_n,
                                            sem_i)
                gns.start()
                gnd.start()

            step(src_c, dst_c, 0, r0, sg0, r1, sg1, ss0, ss1,
                 src_c.at[1], None)
            step(src_c, dst_c, 1, r1, sg1, r0, sg0, ss1, ss0,
                 src_c.at[2], 0)

            def pair(p, carry, src_c=src_c, dst_c=dst_c, r0=r0, sg0=sg0,
                     ss0=ss0, r1=r1, sg1=sg1, ss1=ss1):
                j0 = 2 * p
                step(src_c, dst_c, j0, r0, sg0, r1, sg1, ss0, ss1,
                     src_c.at[j0 + 1], j0 - 1)
                step(src_c, dst_c, j0 + 1, r1, sg1, r0, sg0, ss1, ss0,
                     src_c.at[j0 + 2], j0)
                return carry
            lax.fori_loop(1, K // 2, pair, 0)
            if g + 1 < G:
                gns.wait()
                gnd.wait()
                step(src_c, dst_c, K - 1, r0, sg0, r1, sg1, ss0, ss1,
                     src_n.at[0], K - 2)
            else:
                step(src_c, dst_c, K - 1, r0, sg0, r1, sg1, ss0, ss1,
                     None, K - 2)
                pltpu.make_async_copy(
                    r0, shared_agg.at[dst_c.at[K - 1]], ss0).wait()

        plsc.subcore_barrier()
        pltpu.sync_copy(shared_agg.at[pl.ds(base_r, RPT)],
                        agg_out.at[cid, pl.ds(base_r, RPT)])
        if with_counts:
            pltpu.sync_copy(cnt_v, cnt_out.at[pl.ds(wid * NP, NP)])

    return pl.kernel(
        body, out_type=out_type, mesh=mesh, scratch_types=scratch,
        compiler_params=pltpu.CompilerParams(needs_layout_passes=False))


_sc_agg_counts = _make_sc_agg(True)
_sc_agg = _make_sc_agg(False)


BN = 400


def _dotT(a, w):
    return lax.dot_general(a, w, (((1,), (1,)), ((), ())),
                           preferred_element_type=jnp.float32)


def _dense_xr_body(x_ref, wr_ref, b_ref, o_ref):
    o_ref[...] = _dotT(x_ref[...], wr_ref[...]) + b_ref[...]


_dense_xr = pl.pallas_call(
    _dense_xr_body,
    grid=(N // BN,),
    in_specs=[
        pl.BlockSpec((BN, D), lambda i: (i, 0)),
        pl.BlockSpec((D, D), lambda i: (0, 0)),
        pl.BlockSpec((1, D), lambda i: (0, 0)),
    ],
    out_specs=pl.BlockSpec((BN, D), lambda i: (i, 0)),
    out_shape=jax.ShapeDtypeStruct((N, D), jnp.float32),
)


def _dense0_body(agg_ref, cnt_ref, xr_ref, wl_ref, o_ref):
    agg = agg_ref[0] + agg_ref[1]
    cnt = jnp.sum(cnt_ref[...], axis=1)
    mean = agg / jnp.maximum(cnt, 1.0)[:, None]
    o_ref[...] = jnp.maximum(_dotT(mean, wl_ref[...]) + xr_ref[...], 0.0)


_dense0 = pl.pallas_call(
    _dense0_body,
    grid=(N // BN,),
    in_specs=[
        pl.BlockSpec((NC, BN, D), lambda i: (0, i, 0)),
        pl.BlockSpec((BN, NW), lambda i: (i, 0)),
        pl.BlockSpec((BN, D), lambda i: (i, 0)),
        pl.BlockSpec((D, D), lambda i: (0, 0)),
    ],
    out_specs=pl.BlockSpec((BN, D), lambda i: (i, 0)),
    out_shape=jax.ShapeDtypeStruct((N, D), jnp.float32),
)


def _dense1_body(agg_ref, cnt_ref, xr_ref, wl_ref,
                 wp1_ref, bp1_ref, wp2_ref, bp2_ref, o_ref):
    agg = agg_ref[0] + agg_ref[1]
    cnt = jnp.sum(cnt_ref[...], axis=1)
    mean = agg / jnp.maximum(cnt, 1.0)[:, None]
    h1 = jnp.maximum(_dotT(mean, wl_ref[...]) + xr_ref[...], 0.0)
    p = _dotT(h1, wp1_ref[...]) + bp1_ref[...]
    o_ref[...] = _dotT(p, wp2_ref[...]) + bp2_ref[...]


_dense1 = pl.pallas_call(
    _dense1_body,
    grid=(N // BN,),
    in_specs=[
        pl.BlockSpec((NC, BN, D), lambda i: (0, i, 0)),
        pl.BlockSpec((BN, NW), lambda i: (i, 0)),
        pl.BlockSpec((BN, D), lambda i: (i, 0)),
        pl.BlockSpec((D, D), lambda i: (0, 0)),
        pl.BlockSpec((D, D), lambda i: (0, 0)),
        pl.BlockSpec((1, D), lambda i: (0, 0)),
        pl.BlockSpec((D, D), lambda i: (0, 0)),
        pl.BlockSpec((1, D), lambda i: (0, 0)),
    ],
    out_specs=pl.BlockSpec((BN, D), lambda i: (i, 0)),
    out_shape=jax.ShapeDtypeStruct((N, D), jnp.float32),
)


def kernel(x, edge_index, Wl0, bl0, Wr0, Wl1, bl1, Wr1, Wp1, bp1, Wp2, bp2):
    e = edge_index.reshape(2, NW, G, K, C)
    xr0 = _dense_xr(x, Wr0, bl0.reshape(1, D))
    agg0, cnt_flat = _sc_agg_counts(e, x)
    cnt_t = cnt_flat.reshape(NW, NP).T
    h0 = _dense0(agg0, cnt_t, xr0, Wl0)
    xr1 = _dense_xr(h0, Wr1, bl1.reshape(1, D))
    (agg1,) = _sc_agg(e, h0)
    return _dense1(agg1, cnt_t, xr1, Wl1,
                   Wp1, bp1.reshape(1, D), Wp2, bp2.reshape(1, D))


# speedup vs baseline: 1.0831x; 1.0507x over previous
import jax
import jax.numpy as jnp
from jax import lax
from jax.experimental import pallas as pl
from jax.experimental.pallas import tpu as pltpu
from jax.experimental.pallas import tpu_sc as plsc

N = 10000
E = 320000
D = 128
NC = 2
NS = 16
NW = NC * NS
NP = 10240
EPT = E // NW
C = 80
NCHUNK = EPT // C
G = 5
K = NCHUNK // G
RPT = NP // NS


def _make_sc_agg(with_counts: bool):
    out_type = [jax.ShapeDtypeStruct((NC, NP, D), jnp.float32)]
    scratch = [
        pltpu.VMEM_SHARED((NP, D), jnp.float32),
        pltpu.VMEM((K, C), jnp.int32),
        pltpu.VMEM((K, C), jnp.int32),
        pltpu.VMEM((K, C), jnp.int32),
        pltpu.VMEM((K, C), jnp.int32),
        pltpu.VMEM((C, D), jnp.float32),
        pltpu.VMEM((C, D), jnp.float32),
        pltpu.SemaphoreType.DMA,
        pltpu.SemaphoreType.DMA,
        pltpu.SemaphoreType.DMA,
        pltpu.SemaphoreType.DMA,
        pltpu.SemaphoreType.DMA,
        pltpu.SemaphoreType.DMA,
    ]
    if with_counts:
        out_type.append(jax.ShapeDtypeStruct((NW * NP,), jnp.float32))
        scratch.append(pltpu.VMEM((NP,), jnp.float32))

    mesh = plsc.VectorSubcoreMesh(core_axis_name="c", subcore_axis_name="s")

    def body(e_hbm, x_hbm, agg_out, *rest):
        if with_counts:
            (cnt_out, shared_agg, src0, src1, dst0, dst1, rows_a, rows_b,
             sem_ga, sem_gb, sem_sa, sem_sb, sem_i, sem_z, cnt_v) = rest
        else:
            (shared_agg, src0, src1, dst0, dst1, rows_a, rows_b,
             sem_ga, sem_gb, sem_sa, sem_sb, sem_i, sem_z) = rest
        srcbuf = [src0, src1]
        dstbuf = [dst0, dst1]
        cid = lax.axis_index("c")
        sid = lax.axis_index("s")
        wid = sid * NC + cid

        g0s = pltpu.make_async_copy(e_hbm.at[0, wid, 0], src0, sem_i)
        g0d = pltpu.make_async_copy(e_hbm.at[1, wid, 0], dst0, sem_i)
        g0s.start()
        g0d.start()

        z16 = jnp.zeros((16,), jnp.float32)

        def zrow(r, carry):
            for k in range(D // 16):
                rows_a[r, pl.ds(k * 16, 16)] = z16
                rows_b[r, pl.ds(k * 16, 16)] = z16
            return carry
        lax.fori_loop(0, C, zrow, 0)

        base_r = sid * RPT
        zcps = []
        for j in range(RPT // C):
            zsrc = rows_a if j % 2 == 0 else rows_b
            cp = pltpu.make_async_copy(
                zsrc, shared_agg.at[pl.ds(base_r + j * C, C)], sem_z)
            cp.start()
            zcps.append(cp)

        if with_counts:
            def zcnt(i, carry):
                cnt_v[pl.ds(i * 16, 16)] = z16
                return carry
            lax.fori_loop(0, NP // 16, zcnt, 0)

        for cp in zcps:
            cp.wait()
        g0s.wait()
        g0d.wait()
        pltpu.async_copy(x_hbm.at[src0.at[0]], rows_a, sem_ga)
        plsc.subcore_barrier()

        ones16 = jnp.ones((16,), jnp.float32)

        def counts(dst_i, j):
            if with_counts:
                for k in range(C // 16):
                    idx16 = dst_i[j, pl.ds(k * 16, 16)]
                    plsc.addupdate_scatter(cnt_v, [idx16], ones16)

        def step(src_c, dst_c, j, rows, semg, rows_o, semg_o, sems, sems_o,
                 pre_idx, prev_j):
            pltpu.make_async_copy(x_hbm.at[src_c.at[j]], rows, semg).wait()
            counts(dst_c, j)
            if prev_j is not None:
                pltpu.make_async_copy(
                    rows_o, shared_agg.at[dst_c.at[prev_j]], sems_o).wait()
            if pre_idx is not None:
                pltpu.async_copy(x_hbm.at[pre_idx], rows_o, semg_o)
            pltpu.async_copy(rows, shared_agg.at[dst_c.at[j]], sems, add=True)

        for g in range(G):
            src_c, dst_c = srcbuf[g % 2], dstbuf[g % 2]
            src_n, dst_n = srcbuf[(g + 1) % 2], dstbuf[(g + 1) % 2]
            if g % 2 == 0:
                r0, sg0, ss0 = rows_a, sem_ga, sem_sa
                r1, sg1, ss1 = rows_b, sem_gb, sem_sb
            else:
                r0, sg0, ss0 = rows_b, sem_gb, sem_sb
                r1, sg1, ss1 = rows_a, sem_ga, sem_sa
            if g > 0:
                pltpu.make_async_copy(
                    r1, shared_agg.at[dst_n.at[K - 1]], ss1).wait()
            if g + 1 < G:
                gns = pltpu.make_async_copy(e_hbm.at[0, wid, g + 1], src_n,
                                            sem_i)
                gnd = pltpu.make_async_copy(e_hbm.at[1, wid, g + 1], dst_n,
                                            sem_i)
                gns.start()
                gnd.start()

            step(src_c, dst_c, 0, r0, sg0, r1, sg1, ss0, ss1,
                 src_c.at[1], None)
            step(src_c, dst_c, 1, r1, sg1, r0, sg0, ss1, ss0,
                 src_c.at[2], 0)

            def pair(p, carry, src_c=src_c, dst_c=dst_c, r0=r0, sg0=sg0,
                     ss0=ss0, r1=r1, sg1=sg1, ss1=ss1):
                j0 = 2 * p
                step(src_c, dst_c, j0, r0, sg0, r1, sg1, ss0, ss1,
                     src_c.at[j0 + 1], j0 - 1)
                step(src_c, dst_c, j0 + 1, r1, sg1, r0, sg0, ss1, ss0,
                     src_c.at[j0 + 2], j0)
                return carry
            lax.fori_loop(1, K // 2, pair, 0)
            if g + 1 < G:
                gns.wait()
                gnd.wait()
                step(src_c, dst_c, K - 1, r0, sg0, r1, sg1, ss0, ss1,
                     src_n.at[0], K - 2)
            else:
                step(src_c, dst_c, K - 1, r0, sg0, r1, sg1, ss0, ss1,
                     None, K - 2)
                pltpu.make_async_copy(
                    r0, shared_agg.at[dst_c.at[K - 1]], ss0).wait()

        plsc.subcore_barrier()
        pltpu.sync_copy(shared_agg.at[pl.ds(base_r, RPT)],
                        agg_out.at[cid, pl.ds(base_r, RPT)])
        if with_counts:
            pltpu.sync_copy(cnt_v, cnt_out.at[pl.ds(wid * NP, NP)])

    return pl.kernel(
        body, out_type=out_type, mesh=mesh, scratch_types=scratch,
        compiler_params=pltpu.CompilerParams(needs_layout_passes=False))


_sc_agg_counts = _make_sc_agg(True)
_sc_agg = _make_sc_agg(False)


BN = 1000


def _dotT(a, w):
    return lax.dot_general(a, w, (((1,), (1,)), ((), ())),
                           preferred_element_type=jnp.float32)


def _dense_xr_body(x_ref, wr_ref, b_ref, o_ref):
    o_ref[...] = _dotT(x_ref[...], wr_ref[...]) + b_ref[...]


_dense_xr = pl.pallas_call(
    _dense_xr_body,
    grid=(N // BN,),
    in_specs=[
        pl.BlockSpec((BN, D), lambda i: (i, 0)),
        pl.BlockSpec((D, D), lambda i: (0, 0)),
        pl.BlockSpec((1, D), lambda i: (0, 0)),
    ],
    out_specs=pl.BlockSpec((BN, D), lambda i: (i, 0)),
    out_shape=jax.ShapeDtypeStruct((N, D), jnp.float32),
)


def _dense0_body(agg_ref, cnt_ref, xr_ref, wl_ref, o_ref):
    agg = agg_ref[0] + agg_ref[1]
    cnt = jnp.sum(cnt_ref[...], axis=1)
    mean = agg / jnp.maximum(cnt, 1.0)[:, None]
    o_ref[...] = jnp.maximum(_dotT(mean, wl_ref[...]) + xr_ref[...], 0.0)


_dense0 = pl.pallas_call(
    _dense0_body,
    grid=(N // BN,),
    in_specs=[
        pl.BlockSpec((NC, BN, D), lambda i: (0, i, 0)),
        pl.BlockSpec((BN, NW), lambda i: (i, 0)),
        pl.BlockSpec((BN, D), lambda i: (i, 0)),
        pl.BlockSpec((D, D), lambda i: (0, 0)),
    ],
    out_specs=pl.BlockSpec((BN, D), lambda i: (i, 0)),
    out_shape=jax.ShapeDtypeStruct((N, D), jnp.float32),
)


def _dense1_body(agg_ref, cnt_ref, xr_ref, wl_ref,
                 wp1_ref, bp1_ref, wp2_ref, bp2_ref, o_ref):
    agg = agg_ref[0] + agg_ref[1]
    cnt = jnp.sum(cnt_ref[...], axis=1)
    mean = agg / jnp.maximum(cnt, 1.0)[:, None]
    h1 = jnp.maximum(_dotT(mean, wl_ref[...]) + xr_ref[...], 0.0)
    p = _dotT(h1, wp1_ref[...]) + bp1_ref[...]
    o_ref[...] = _dotT(p, wp2_ref[...]) + bp2_ref[...]


_dense1 = pl.pallas_call(
    _dense1_body,
    grid=(N // BN,),
    in_specs=[
        pl.BlockSpec((NC, BN, D), lambda i: (0, i, 0)),
        pl.BlockSpec((BN, NW), lambda i: (i, 0)),
        pl.BlockSpec((BN, D), lambda i: (i, 0)),
        pl.BlockSpec((D, D), lambda i: (0, 0)),
        pl.BlockSpec((D, D), lambda i: (0, 0)),
        pl.BlockSpec((1, D), lambda i: (0, 0)),
        pl.BlockSpec((D, D), lambda i: (0, 0)),
        pl.BlockSpec((1, D), lambda i: (0, 0)),
    ],
    out_specs=pl.BlockSpec((BN, D), lambda i: (i, 0)),
    out_shape=jax.ShapeDtypeStruct((N, D), jnp.float32),
)


def kernel(x, edge_index, Wl0, bl0, Wr0, Wl1, bl1, Wr1, Wp1, bp1, Wp2, bp2):
    e = edge_index.reshape(2, NW, G, K, C)
    xr0 = _dense_xr(x, Wr0, bl0.reshape(1, D))
    agg0, cnt_flat = _sc_agg_counts(e, x)
    cnt_t = cnt_flat.reshape(NW, NP).T
    h0 = _dense0(agg0, cnt_t, xr0, Wl0)
    xr1 = _dense_xr(h0, Wr1, bl1.reshape(1, D))
    (agg1,) = _sc_agg(e, h0)
    return _dense1(agg1, cnt_t, xr1, Wl1,
                   Wp1, bp1.reshape(1, D), Wp2, bp2.reshape(1, D))


# speedup vs baseline: 1.1011x; 1.0167x over previous
import jax
import jax.numpy as jnp
from jax import lax
from jax.experimental import pallas as pl
from jax.experimental.pallas import tpu as pltpu
from jax.experimental.pallas import tpu_sc as plsc

N = 10000
E = 320000
D = 128
NC = 2
NS = 16
NW = NC * NS
NP = 10240
EPT = E // NW
C = 80
NCHUNK = EPT // C
G = 5
K = NCHUNK // G
RPT = NP // NS


def _make_sc_agg(with_counts: bool):
    out_type = [jax.ShapeDtypeStruct((NC, NP, D), jnp.float32)]
    scratch = [
        pltpu.VMEM_SHARED((NP, D), jnp.float32),
        pltpu.VMEM((K, C), jnp.int32),
        pltpu.VMEM((K, C), jnp.int32),
        pltpu.VMEM((K, C), jnp.int32),
        pltpu.VMEM((K, C), jnp.int32),
        pltpu.VMEM((C, D), jnp.float32),
        pltpu.VMEM((C, D), jnp.float32),
        pltpu.SemaphoreType.DMA,
        pltpu.SemaphoreType.DMA,
        pltpu.SemaphoreType.DMA,
        pltpu.SemaphoreType.DMA,
        pltpu.SemaphoreType.DMA,
        pltpu.SemaphoreType.DMA,
    ]
    if with_counts:
        out_type.append(jax.ShapeDtypeStruct((NW * NP,), jnp.float32))
        scratch.append(pltpu.VMEM((NP,), jnp.float32))

    mesh = plsc.VectorSubcoreMesh(core_axis_name="c", subcore_axis_name="s")

    def body(e_hbm, x_hbm, agg_out, *rest):
        if with_counts:
            (cnt_out, shared_agg, src0, src1, dst0, dst1, rows_a, rows_b,
             sem_ga, sem_gb, sem_sa, sem_sb, sem_i, sem_z, cnt_v) = rest
        else:
            (shared_agg, src0, src1, dst0, dst1, rows_a, rows_b,
             sem_ga, sem_gb, sem_sa, sem_sb, sem_i, sem_z) = rest
        srcbuf = [src0, src1]
        dstbuf = [dst0, dst1]
        cid = lax.axis_index("c")
        sid = lax.axis_index("s")
        wid = sid * NC + cid

        g0s = pltpu.make_async_copy(e_hbm.at[0, wid, 0], src0, sem_i)
        g0d = pltpu.make_async_copy(e_hbm.at[1, wid, 0], dst0, sem_i)
        g0s.start()
        g0d.start()

        z16 = jnp.zeros((16,), jnp.float32)

        def zrow(r, carry):
            for k in range(D // 16):
                rows_a[r, pl.ds(k * 16, 16)] = z16
                rows_b[r, pl.ds(k * 16, 16)] = z16
            return carry
        lax.fori_loop(0, C, zrow, 0)

        base_r = sid * RPT
        zcps = []
        for j in range(RPT // C):
            zsrc = rows_a if j % 2 == 0 else rows_b
            cp = pltpu.make_async_copy(
                zsrc, shared_agg.at[pl.ds(base_r + j * C, C)], sem_z)
            cp.start()
            zcps.append(cp)

        if with_counts:
            def zcnt(i, carry):
                cnt_v[pl.ds(i * 16, 16)] = z16
                return carry
            lax.fori_loop(0, NP // 16, zcnt, 0)

        for cp in zcps:
            cp.wait()
        g0s.wait()
        g0d.wait()
        pltpu.async_copy(x_hbm.at[src0.at[0]], rows_a, sem_ga)
        plsc.subcore_barrier()

        ones16 = jnp.ones((16,), jnp.float32)

        def counts(dst_i, j):
            if with_counts:
                for k in range(C // 16):
                    idx16 = dst_i[j, pl.ds(k * 16, 16)]
                    plsc.addupdate_scatter(cnt_v, [idx16], ones16)

        def step(src_c, dst_c, j, rows, semg, rows_o, semg_o, sems, sems_o,
                 pre_idx, prev_j):
            pltpu.make_async_copy(x_hbm.at[src_c.at[j]], rows, semg).wait()
            counts(dst_c, j)
            if prev_j is not None:
                pltpu.make_async_copy(
                    rows_o, shared_agg.at[dst_c.at[prev_j]], sems_o).wait()
            if pre_idx is not None:
                pltpu.async_copy(x_hbm.at[pre_idx], rows_o, semg_o)
            pltpu.async_copy(rows, shared_agg.at[dst_c.at[j]], sems, add=True)

        for g in range(G):
            src_c, dst_c = srcbuf[g % 2], dstbuf[g % 2]
            src_n, dst_n = srcbuf[(g + 1) % 2], dstbuf[(g + 1) % 2]
            if g % 2 == 0:
                r0, sg0, ss0 = rows_a, sem_ga, sem_sa
                r1, sg1, ss1 = rows_b, sem_gb, sem_sb
            else:
                r0, sg0, ss0 = rows_b, sem_gb, sem_sb
                r1, sg1, ss1 = rows_a, sem_ga, sem_sa
            if g > 0:
                pltpu.make_async_copy(
                    r1, shared_agg.at[dst_n.at[K - 1]], ss1).wait()
            if g + 1 < G:
                gns = pltpu.make_async_copy(e_hbm.at[0, wid, g + 1], src_n,
                                            sem_i)
                gnd = pltpu.make_async_copy(e_hbm.at[1, wid, g + 1], dst_n,
                                            sem_i)
                gns.start()
                gnd.start()

            step(src_c, dst_c, 0, r0, sg0, r1, sg1, ss0, ss1,
                 src_c.at[1], None)
            step(src_c, dst_c, 1, r1, sg1, r0, sg0, ss1, ss0,
                 src_c.at[2], 0)

            def pair(p, carry, src_c=src_c, dst_c=dst_c, r0=r0, sg0=sg0,
                     ss0=ss0, r1=r1, sg1=sg1, ss1=ss1):
                j0 = 2 * p
                step(src_c, dst_c, j0, r0, sg0, r1, sg1, ss0, ss1,
                     src_c.at[j0 + 1], j0 - 1)
                step(src_c, dst_c, j0 + 1, r1, sg1, r0, sg0, ss1, ss0,
                     src_c.at[j0 + 2], j0)
                return carry
            lax.fori_loop(1, K // 2, pair, 0)
            if g + 1 < G:
                gns.wait()
                gnd.wait()
                step(src_c, dst_c, K - 1, r0, sg0, r1, sg1, ss0, ss1,
                     src_n.at[0], K - 2)
            else:
                step(src_c, dst_c, K - 1, r0, sg0, r1, sg1, ss0, ss1,
                     None, K - 2)
                pltpu.make_async_copy(
                    r0, shared_agg.at[dst_c.at[K - 1]], ss0).wait()

        plsc.subcore_barrier()
        pltpu.sync_copy(shared_agg.at[pl.ds(base_r, RPT)],
                        agg_out.at[cid, pl.ds(base_r, RPT)])
        if with_counts:
            pltpu.sync_copy(cnt_v, cnt_out.at[pl.ds(wid * NP, NP)])

    return pl.kernel(
        body, out_type=out_type, mesh=mesh, scratch_types=scratch,
        compiler_params=pltpu.CompilerParams(needs_layout_passes=False))


_sc_agg_counts = _make_sc_agg(True)
_sc_agg = _make_sc_agg(False)


BN = 2000


def _dotT(a, w):
    return lax.dot_general(a, w, (((1,), (1,)), ((), ())),
                           preferred_element_type=jnp.float32)


def _dense_xr_body(x_ref, wr_ref, b_ref, o_ref):
    o_ref[...] = _dotT(x_ref[...], wr_ref[...]) + b_ref[...]


_dense_xr = pl.pallas_call(
    _dense_xr_body,
    grid=(N // BN,),
    in_specs=[
        pl.BlockSpec((BN, D), lambda i: (i, 0)),
        pl.BlockSpec((D, D), lambda i: (0, 0)),
        pl.BlockSpec((1, D), lambda i: (0, 0)),
    ],
    out_specs=pl.BlockSpec((BN, D), lambda i: (i, 0)),
    out_shape=jax.ShapeDtypeStruct((N, D), jnp.float32),
)


def _dense0_body(agg_ref, cnt_ref, xr_ref, wl_ref, o_ref):
    agg = agg_ref[0] + agg_ref[1]
    cnt = jnp.sum(cnt_ref[...], axis=1)
    mean = agg / jnp.maximum(cnt, 1.0)[:, None]
    o_ref[...] = jnp.maximum(_dotT(mean, wl_ref[...]) + xr_ref[...], 0.0)


_dense0 = pl.pallas_call(
    _dense0_body,
    grid=(N // BN,),
    in_specs=[
        pl.BlockSpec((NC, BN, D), lambda i: (0, i, 0)),
        pl.BlockSpec((BN, NW), lambda i: (i, 0)),
        pl.BlockSpec((BN, D), lambda i: (i, 0)),
        pl.BlockSpec((D, D), lambda i: (0, 0)),
    ],
    out_specs=pl.BlockSpec((BN, D), lambda i: (i, 0)),
    out_shape=jax.ShapeDtypeStruct((N, D), jnp.float32),
)


def _dense1_body(agg_ref, cnt_ref, xr_ref, wl_ref,
                 wp1_ref, bp1_ref, wp2_ref, bp2_ref, o_ref):
    agg = agg_ref[0] + agg_ref[1]
    cnt = jnp.sum(cnt_ref[...], axis=1)
    mean = agg / jnp.maximum(cnt, 1.0)[:, None]
    h1 = jnp.maximum(_dotT(mean, wl_ref[...]) + xr_ref[...], 0.0)
    p = _dotT(h1, wp1_ref[...]) + bp1_ref[...]
    o_ref[...] = _dotT(p, wp2_ref[...]) + bp2_ref[...]


_dense1 = pl.pallas_call(
    _dense1_body,
    grid=(N // BN,),
    in_specs=[
        pl.BlockSpec((NC, BN, D), lambda i: (0, i, 0)),
        pl.BlockSpec((BN, NW), lambda i: (i, 0)),
        pl.BlockSpec((BN, D), lambda i: (i, 0)),
        pl.BlockSpec((D, D), lambda i: (0, 0)),
        pl.BlockSpec((D, D), lambda i: (0, 0)),
        pl.BlockSpec((1, D), lambda i: (0, 0)),
        pl.BlockSpec((D, D), lambda i: (0, 0)),
        pl.BlockSpec((1, D), lambda i: (0, 0)),
    ],
    out_specs=pl.BlockSpec((BN, D), lambda i: (i, 0)),
    out_shape=jax.ShapeDtypeStruct((N, D), jnp.float32),
)


def kernel(x, edge_index, Wl0, bl0, Wr0, Wl1, bl1, Wr1, Wp1, bp1, Wp2, bp2):
    e = edge_index.reshape(2, NW, G, K, C)
    xr0 = _dense_xr(x, Wr0, bl0.reshape(1, D))
    agg0, cnt_flat = _sc_agg_counts(e, x)
    cnt_t = cnt_flat.reshape(NW, NP).T
    h0 = _dense0(agg0, cnt_t, xr0, Wl0)
    xr1 = _dense_xr(h0, Wr1, bl1.reshape(1, D))
    (agg1,) = _sc_agg(e, h0)
    return _dense1(agg1, cnt_t, xr1, Wl1,
                   Wp1, bp1.reshape(1, D), Wp2, bp2.reshape(1, D))
